# Initial kernel scaffold; baseline (speedup 1.0000x reference)
import jax
import jax.numpy as jnp
from jax.experimental import pallas as pl


def kernel(x, params_packed):
    raise NotImplementedError("write your pallas kernel here")


# speedup vs baseline: 3.2556x; 3.2556x over previous
import functools

import jax
import jax.numpy as jnp
from jax.experimental import pallas as pl
from jax.experimental.pallas import tpu as pltpu

_IN_F, _HID_F, _OUT_F = 4, 50, 3
_W2_ROW = 16
_B2_ROW = 144
_OUT_W = 8


def _mlp_kernel(x_ref, w1_ref, b1_ref, w2_ref, b2_ref, o_ref):
    h = jnp.dot(x_ref[...], w1_ref[...], preferred_element_type=jnp.float32)
    h = jnp.maximum(h + b1_ref[...], 0.0)
    y = jnp.dot(h, w2_ref[...], preferred_element_type=jnp.float32)
    o_ref[...] = y + b2_ref[...]


@functools.partial(jax.jit, static_argnames=("tile_b",))
def _forward(x, params_packed, tile_b=2048):
    B = x.shape[0]
    tb = min(tile_b, max(8, ((B + 7) // 8) * 8))
    b_pad = ((B + tb - 1) // tb) * tb
    if b_pad != B:
        x = jnp.pad(x, ((0, b_pad - B), (0, 0)))

    w1 = params_packed[0:_IN_F, :]
    b1 = params_packed[8:9, :]
    w2 = params_packed[_W2_ROW:_W2_ROW + 128, :_OUT_W]
    b2 = params_packed[_B2_ROW:_B2_ROW + 1, :_OUT_W]

    grid = (b_pad // tb,)
    cost = pl.CostEstimate(
        flops=2 * b_pad * (_IN_F * 128 + 128 * _OUT_W),
        transcendentals=0,
        bytes_accessed=(b_pad * _IN_F + 152 * 128 + b_pad * _OUT_W) * 4,
    )
    out = pl.pallas_call(
        _mlp_kernel,
        out_shape=jax.ShapeDtypeStruct((b_pad, _OUT_W), jnp.float32),
        grid=grid,
        in_specs=[
            pl.BlockSpec((tb, _IN_F), lambda i: (i, 0)),
            pl.BlockSpec((_IN_F, 128), lambda i: (0, 0)),
            pl.BlockSpec((1, 128), lambda i: (0, 0)),
            pl.BlockSpec((128, _OUT_W), lambda i: (0, 0)),
            pl.BlockSpec((1, _OUT_W), lambda i: (0, 0)),
        ],
        out_specs=pl.BlockSpec((tb, _OUT_W), lambda i: (i, 0)),
        compiler_params=pltpu.CompilerParams(
            dimension_semantics=("parallel",),
        ),
        cost_estimate=cost,
    )(x, w1, b1, w2, b2)
    return out[:B, :_OUT_F]


def kernel(x, params_packed):
    return _forward(x, params_packed)


# speedup vs baseline: 24.3056x; 7.4658x over previous
import functools

import jax
import jax.numpy as jnp
from jax.experimental import pallas as pl
from jax.experimental.pallas import tpu as pltpu

_IN_F, _HID_F, _OUT_F = 4, 50, 3
_W2_ROW = 16
_B2_ROW = 144
_OUT_W = 8


def _mlp_kernel_t(xt_ref, w1_ref, b1t_ref, w2_ref, o_ref):
    ht = jax.lax.dot_general(
        w1_ref[...], xt_ref[...],
        dimension_numbers=(((0,), (0,)), ((), ())),
        preferred_element_type=jnp.float32,
    )
    ht = jnp.maximum(ht + b1t_ref[...], 0.0)
    yt = jax.lax.dot_general(
        w2_ref[...], ht,
        dimension_numbers=(((0,), (0,)), ((), ())),
        preferred_element_type=jnp.float32,
    )
    o_ref[...] = yt[:_OUT_F, :]


@functools.partial(jax.jit, static_argnames=("tile_n",))
def _forward(x, params_packed, tile_n=8192):
    B = x.shape[0]
    xt = x.T
    tn = min(tile_n, max(128, -(-B // 128) * 128))
    n_pad = -(-B // tn) * tn
    if n_pad != B:
        xt = jnp.pad(xt, ((0, 0), (0, n_pad - B)))

    w1 = params_packed[0:_IN_F, :]
    b1 = params_packed[8:9, :]
    b2 = params_packed[_B2_ROW:_B2_ROW + 1, :_OUT_W]
    w2 = params_packed[_W2_ROW:_W2_ROW + 128, :_OUT_W]
    b1 = b1.at[0, _HID_F].set(1.0)
    w2 = w2.at[_HID_F, :].set(b2[0, :])
    b1t = b1.T

    grid = (n_pad // tn,)
    cost = pl.CostEstimate(
        flops=2 * n_pad * (_IN_F * 128 + 128 * _OUT_W),
        transcendentals=0,
        bytes_accessed=(n_pad * _IN_F + 152 * 128 + n_pad * _OUT_F) * 4,
    )
    out_t = pl.pallas_call(
        _mlp_kernel_t,
        out_shape=jax.ShapeDtypeStruct((_OUT_F, n_pad), jnp.float32),
        grid=grid,
        in_specs=[
            pl.BlockSpec((_IN_F, tn), lambda i: (0, i)),
            pl.BlockSpec((_IN_F, 128), lambda i: (0, 0)),
            pl.BlockSpec((128, 1), lambda i: (0, 0)),
            pl.BlockSpec((128, _OUT_W), lambda i: (0, 0)),
        ],
        out_specs=pl.BlockSpec((_OUT_F, tn), lambda i: (0, i)),
        compiler_params=pltpu.CompilerParams(
            dimension_semantics=("parallel",),
        ),
        cost_estimate=cost,
    )(xt, w1, b1t, w2)
    return out_t[:, :B].T


def kernel(x, params_packed):
    return _forward(x, params_packed)


# speedup vs baseline: 30.8296x; 1.2684x over previous
import functools

import jax
import jax.numpy as jnp
from jax.experimental import pallas as pl
from jax.experimental.pallas import tpu as pltpu

_IN_F, _HID_F, _OUT_F = 4, 50, 3
_W2_ROW = 16
_B2_ROW = 144
_OUT_W = 8
_HID_W = 64


def _mlp_kernel_t(xt_ref, w1_ref, b1t_ref, w2_ref, o_ref):
    ht = jax.lax.dot_general(
        w1_ref[...], xt_ref[...],
        dimension_numbers=(((0,), (0,)), ((), ())),
        preferred_element_type=jnp.float32,
    )
    ht = jnp.maximum(ht + b1t_ref[...], 0.0)
    yt = jax.lax.dot_general(
        w2_ref[...], ht,
        dimension_numbers=(((0,), (0,)), ((), ())),
        preferred_element_type=jnp.float32,
    )
    o_ref[...] = yt[:_OUT_F, :]


@functools.partial(jax.jit, static_argnames=("tile_n",))
def _forward(x, params_packed, tile_n=8192):
    B = x.shape[0]
    xt = x.T
    tn = min(tile_n, max(128, -(-B // 128) * 128))
    n_pad = -(-B // tn) * tn
    if n_pad != B:
        xt = jnp.pad(xt, ((0, 0), (0, n_pad - B)))

    w1 = params_packed[0:_IN_F, :_HID_W]
    b1 = params_packed[8:9, :_HID_W]
    b2 = params_packed[_B2_ROW:_B2_ROW + 1, :_OUT_W]
    w2 = params_packed[_W2_ROW:_W2_ROW + _HID_W, :_OUT_W]
    b1 = b1.at[0, _HID_F].set(1.0)
    w2 = w2.at[_HID_F, :].set(b2[0, :])
    b1t = b1.T

    grid = (n_pad // tn,)
    cost = pl.CostEstimate(
        flops=2 * n_pad * (_IN_F * _HID_W + _HID_W * _OUT_W),
        transcendentals=0,
        bytes_accessed=(n_pad * _IN_F + 152 * 128 + n_pad * _OUT_F) * 4,
    )
    out_t = pl.pallas_call(
        _mlp_kernel_t,
        out_shape=jax.ShapeDtypeStruct((_OUT_F, n_pad), jnp.float32),
        grid=grid,
        in_specs=[
            pl.BlockSpec((_IN_F, tn), lambda i: (0, i)),
            pl.BlockSpec((_IN_F, _HID_W), lambda i: (0, 0)),
            pl.BlockSpec((_HID_W, 1), lambda i: (0, 0)),
            pl.BlockSpec((_HID_W, _OUT_W), lambda i: (0, 0)),
        ],
        out_specs=pl.BlockSpec((_OUT_F, tn), lambda i: (0, i)),
        compiler_params=pltpu.CompilerParams(
            dimension_semantics=("parallel",),
        ),
        cost_estimate=cost,
    )(xt, w1, b1t, w2)
    return out_t[:, :B].T


def kernel(x, params_packed):
    return _forward(x, params_packed)


# speedup vs baseline: 51.0432x; 1.6557x over previous
import functools

import jax
import jax.numpy as jnp
from jax.experimental import pallas as pl
from jax.experimental.pallas import tpu as pltpu

_IN_F, _HID_F, _OUT_F = 4, 50, 3
_W2_ROW = 16
_B2_ROW = 144
_OUT_W = 8
_HID_W = 64


def _mlp_kernel_t(xt_ref, w1_ref, b1t_ref, w2_ref, o_ref):
    ht = jax.lax.dot_general(
        w1_ref[...], xt_ref[...],
        dimension_numbers=(((0,), (0,)), ((), ())),
        preferred_element_type=jnp.float32,
    )
    ht = jnp.maximum(ht + b1t_ref[...], 0.0)
    yt = jax.lax.dot_general(
        w2_ref[...], ht,
        dimension_numbers=(((0,), (0,)), ((), ())),
        preferred_element_type=jnp.float32,
    )
    o_ref[...] = yt[:_OUT_F, :]


@functools.partial(jax.jit, static_argnames=("tile_n",))
def _forward(x, params_packed, tile_n=32768):
    B = x.shape[0]
    xt = x.T
    tn = min(tile_n, max(128, -(-B // 128) * 128))
    n_pad = -(-B // tn) * tn
    if n_pad != B:
        xt = jnp.pad(xt, ((0, 0), (0, n_pad - B)))

    w1 = params_packed[0:_IN_F, :_HID_W]
    b1 = params_packed[8:9, :_HID_W]
    b2 = params_packed[_B2_ROW:_B2_ROW + 1, :_OUT_W]
    w2 = params_packed[_W2_ROW:_W2_ROW + _HID_W, :_OUT_W]
    b1 = b1.at[0, _HID_F].set(1.0)
    w2 = w2.at[_HID_F, :].set(b2[0, :])
    b1t = b1.T

    grid = (n_pad // tn,)
    cost = pl.CostEstimate(
        flops=2 * n_pad * (_IN_F * _HID_W + _HID_W * _OUT_W),
        transcendentals=0,
        bytes_accessed=(n_pad * _IN_F + 152 * 128 + n_pad * _OUT_F) * 4,
    )
    out_t = pl.pallas_call(
        _mlp_kernel_t,
        out_shape=jax.ShapeDtypeStruct((_OUT_F, n_pad), jnp.float32),
        grid=grid,
        in_specs=[
            pl.BlockSpec((_IN_F, tn), lambda i: (0, i)),
            pl.BlockSpec((_IN_F, _HID_W), lambda i: (0, 0)),
            pl.BlockSpec((_HID_W, 1), lambda i: (0, 0)),
            pl.BlockSpec((_HID_W, _OUT_W), lambda i: (0, 0)),
        ],
        out_specs=pl.BlockSpec((_OUT_F, tn), lambda i: (0, i)),
        compiler_params=pltpu.CompilerParams(
            dimension_semantics=("parallel",),
        ),
        cost_estimate=cost,
    )(xt, w1, b1t, w2)
    return out_t[:, :B].T


def kernel(x, params_packed):
    return _forward(x, params_packed)


# speedup vs baseline: 54.0360x; 1.0586x over previous
import functools

import jax
import jax.numpy as jnp
from jax.experimental import pallas as pl
from jax.experimental.pallas import tpu as pltpu

_IN_F, _HID_F, _OUT_F = 4, 50, 3
_W2_ROW = 16
_B2_ROW = 144
_OUT_W = 8
_HID_W = 64


def _mlp_kernel_t(xt_ref, w1_ref, b1t_ref, w2_ref, o_ref):
    ht = jax.lax.dot_general(
        w1_ref[...], xt_ref[...],
        dimension_numbers=(((0,), (0,)), ((), ())),
        preferred_element_type=jnp.float32,
    )
    ht = jnp.maximum(ht + b1t_ref[...], 0.0)
    yt = jax.lax.dot_general(
        w2_ref[...], ht,
        dimension_numbers=(((0,), (0,)), ((), ())),
        preferred_element_type=jnp.float32,
    )
    o_ref[...] = yt[:_OUT_F, :]


@functools.partial(jax.jit, static_argnames=("tile_n",))
def _forward(x, params_packed, tile_n=65536):
    B = x.shape[0]
    xt = x.T
    tn = min(tile_n, max(128, -(-B // 128) * 128))
    n_pad = -(-B // tn) * tn
    if n_pad != B:
        xt = jnp.pad(xt, ((0, 0), (0, n_pad - B)))

    w1 = params_packed[0:_IN_F, :_HID_W]
    b1 = params_packed[8:9, :_HID_W]
    b2 = params_packed[_B2_ROW:_B2_ROW + 1, :_OUT_W]
    w2 = params_packed[_W2_ROW:_W2_ROW + _HID_W, :_OUT_W]
    b1 = b1.at[0, _HID_F].set(1.0)
    w2 = w2.at[_HID_F, :].set(b2[0, :])
    b1t = b1.T

    grid = (n_pad // tn,)
    cost = pl.CostEstimate(
        flops=2 * n_pad * (_IN_F * _HID_W + _HID_W * _OUT_W),
        transcendentals=0,
        bytes_accessed=(n_pad * _IN_F + 152 * 128 + n_pad * _OUT_F) * 4,
    )
    out_t = pl.pallas_call(
        _mlp_kernel_t,
        out_shape=jax.ShapeDtypeStruct((_OUT_F, n_pad), jnp.float32),
        grid=grid,
        in_specs=[
            pl.BlockSpec((_IN_F, tn), lambda i: (0, i)),
            pl.BlockSpec((_IN_F, _HID_W), lambda i: (0, 0)),
            pl.BlockSpec((_HID_W, 1), lambda i: (0, 0)),
            pl.BlockSpec((_HID_W, _OUT_W), lambda i: (0, 0)),
        ],
        out_specs=pl.BlockSpec((_OUT_F, tn), lambda i: (0, i)),
        compiler_params=pltpu.CompilerParams(
            dimension_semantics=("parallel",),
        ),
        cost_estimate=cost,
    )(xt, w1, b1t, w2)
    return out_t[:, :B].T


def kernel(x, params_packed):
    return _forward(x, params_packed)
